# baseline (speedup 1.0000x reference)
import functools

import jax
import jax.numpy as jnp
from jax import lax
from jax.experimental import pallas as pl
from jax.experimental.pallas import tpu as pltpu
from jax.experimental.pallas import tpu_sc as plsc

N = 10000
E = 160000
H = 256
HH = 128
NS = 16
NPAD = 10240
RPT = NPAD // NS
C = 64
NBUF = 4
PD = 3
NCH = 168
ET = NCH * C
EP = NS * ET
RB = 640
GRB = NPAD // RB


def _rsqrt_v(x):
    i = plsc.bitcast(x, jnp.int32)
    y = plsc.bitcast(jnp.int32(0x5F3759DF) - (i >> 1), jnp.float32)
    for _ in range(3):
        y = y * (1.5 - 0.5 * x * y * y)
    return y


def _dis_body(pk_hbm, dis_hbm, ib, idxc, ewt, deg_t, deg_sh):
    c = lax.axis_index("c")
    s = lax.axis_index("s")

    @pl.when(c == 0)
    def _():
        def zb(i, _):
            deg_t[pl.ds(i * 16, 16)] = jnp.zeros((16,), jnp.float32)
            return 0
        lax.fori_loop(0, RPT // 16, zb, 0)
        pltpu.sync_copy(deg_t, deg_sh.at[pl.ds(s * RPT, RPT)])
        plsc.subcore_barrier()

        def chunk(k, _):
            sk = s * NCH + k
            pltpu.sync_copy(pk_hbm.at[pl.ds(sk * 3 * C, 3 * C)], ib)

            def grp(g, _):
                sl = pl.ds(g * 16, 16)
                idxc[sl] = ib[pl.ds(C + g * 16, 16)]
                ewt[sl] = plsc.bitcast(ib[pl.ds(2 * C + g * 16, 16)],
                                       jnp.float32)
                return 0
            lax.fori_loop(0, C // 16, grp, 0)
            pltpu.sync_copy(ewt, deg_sh.at[idxc], add=True)
            return 0
        lax.fori_loop(0, NCH, chunk, 0)
        plsc.subcore_barrier()

        pltpu.sync_copy(deg_sh.at[pl.ds(s * RPT, RPT)], deg_t)

        def rs(i, _):
            sl = pl.ds(i * 16, 16)
            deg_t[sl] = _rsqrt_v(deg_t[sl])
            return 0
        lax.fori_loop(0, RPT // 16, rs, 0)
        pltpu.sync_copy(deg_t, dis_hbm.at[pl.ds(s * RPT, RPT)])


_dis_call = pl.kernel(
    _dis_body,
    out_type=jax.ShapeDtypeStruct((NPAD,), jnp.float32),
    mesh=plsc.VectorSubcoreMesh(core_axis_name="c", subcore_axis_name="s"),
    scratch_types=[
        pltpu.VMEM((3 * C,), jnp.int32),
        pltpu.VMEM((C,), jnp.int32),
        pltpu.VMEM((C,), jnp.float32),
        pltpu.VMEM((RPT,), jnp.float32),
        pltpu.VMEM_SHARED((NPAD,), jnp.float32),
    ],
    compiler_params=pltpu.CompilerParams(needs_layout_passes=False),
)


def _edge_body(h_hbm, pk_hbm, dis_hbm, zero_hbm, acc_hbm, dis_t, *sc):
    c = lax.axis_index("c")
    s = lax.axis_index("s")
    ibs = sc[0:NBUF]
    gis = sc[NBUF:2 * NBUF]
    cis = sc[2 * NBUF:3 * NBUF]
    cfs = sc[3 * NBUF:4 * NBUF]
    gbs = sc[4 * NBUF:5 * NBUF]
    acc_sh = sc[5 * NBUF]
    isems = sc[5 * NBUF + 1:5 * NBUF + 1 + NBUF]
    gsems = sc[5 * NBUF + 1 + NBUF:5 * NBUF + 1 + 2 * NBUF]
    ssems = sc[5 * NBUF + 1 + 2 * NBUF:5 * NBUF + 1 + 3 * NBUF]

    pltpu.sync_copy(dis_hbm, dis_t)
    pltpu.sync_copy(zero_hbm.at[pl.ds(s * RPT, RPT)],
                    acc_sh.at[pl.ds(s * RPT, RPT)])
    plsc.subcore_barrier()

    def idx_copy(k, b):
        return pltpu.make_async_copy(
            pk_hbm.at[pl.ds((s * NCH + k) * 3 * C, 3 * C)], ibs[b], isems[b])

    def prep(b):
        ib, gi, ci, cf = ibs[b], gis[b], cis[b], cfs[b]

        def grp(g, _):
            sl = pl.ds(g * 16, 16)
            rv = ib[pl.ds(g * 16, 16)]
            cv = ib[pl.ds(C + g * 16, 16)]
            wv = plsc.bitcast(ib[pl.ds(2 * C + g * 16, 16)], jnp.float32)
            dr = plsc.load_gather(dis_t, [rv])
            dc = plsc.load_gather(dis_t, [cv])
            cf[sl] = wv * dr * dc
            gi[sl] = rv + c * NPAD
            ci[sl] = cv
            return 0
        lax.fori_loop(0, C // 16, grp, 0)

    def gather_copy(b):
        return pltpu.make_async_copy(h_hbm.at[gis[b]], gbs[b], gsems[b])

    def scale(b):
        gb, cf = gbs[b], cfs[b]

        def grp2(g, _):
            for j in range(16):
                i = g * 16 + j
                cb = plsc.load_gather(cf, [jnp.broadcast_to(i, (16,))])
                for f in range(HH // 16):
                    fs = pl.ds(f * 16, 16)
                    gb[i, fs] = gb[i, fs] * cb
            return 0
        lax.fori_loop(0, C // 16, grp2, 0)

    def scatter_copy(b):
        return pltpu.async_copy(gbs[b], acc_sh.at[cis[b]], ssems[b], add=True)

    def scatter_wait(b):
        pltpu.make_async_copy(gbs[b], acc_sh.at[cis[b]], ssems[b]).wait()

    D = PD
    for j in range(D):
        idx_copy(j, j).start()
    for j in range(D):
        idx_copy(j, j).wait()
        prep(j)
        gather_copy(j).start()
    idx_copy(D, D).start()

    def outer(t, _):
        for b in range(NBUF):
            k = t * NBUF + b
            gather_copy(b).wait()
            scale(b)
            scatter_copy(b)
            nb = (b + D) % NBUF

            @pl.when(k + D < NCH)
            def _():
                @pl.when(k >= NBUF - D)
                def _():
                    scatter_wait(nb)
                idx_copy(k + D, nb).wait()
                prep(nb)
                gather_copy(nb).start()

                nb2 = (b + D + 1) % NBUF

                @pl.when(k + D + 1 < NCH)
                def _():
                    idx_copy(k + D + 1, nb2).start()
        return 0
    lax.fori_loop(0, NCH // NBUF, outer, 0)
    for j in range(NCH - D, NCH):
        scatter_wait(j % NBUF)
    plsc.subcore_barrier()

    pltpu.sync_copy(acc_sh.at[pl.ds(s * RPT, RPT)],
                    acc_hbm.at[pl.ds(c * NPAD + s * RPT, RPT)])


_edge_call = pl.kernel(
    _edge_body,
    out_type=jax.ShapeDtypeStruct((2 * NPAD, HH), jnp.float32),
    mesh=plsc.VectorSubcoreMesh(core_axis_name="c", subcore_axis_name="s"),
    scratch_types=(
        [pltpu.VMEM((NPAD,), jnp.float32)]
        + [pltpu.VMEM((3 * C,), jnp.int32) for _ in range(NBUF)]
        + [pltpu.VMEM((C,), jnp.int32) for _ in range(NBUF)]
        + [pltpu.VMEM((C,), jnp.int32) for _ in range(NBUF)]
        + [pltpu.VMEM((C,), jnp.float32) for _ in range(NBUF)]
        + [pltpu.VMEM((C, HH), jnp.float32) for _ in range(NBUF)]
        + [pltpu.VMEM_SHARED((NPAD, HH), jnp.float32)]
        + [pltpu.SemaphoreType.DMA for _ in range(3 * NBUF)]
    ),
    compiler_params=pltpu.CompilerParams(needs_layout_passes=False),
)


def _mm_first_body(x_ref, w_ref, o_ref):
    o_ref[...] = jnp.dot(x_ref[...], w_ref[...],
                         preferred_element_type=jnp.float32)


_mm_first = pl.pallas_call(
    _mm_first_body,
    grid=(GRB, 2),
    in_specs=[
        pl.BlockSpec((RB, H), lambda i, j: (i, 0)),
        pl.BlockSpec((H, HH), lambda i, j: (0, j)),
    ],
    out_specs=pl.BlockSpec((RB, HH), lambda i, j: (i + GRB * j, 0)),
    out_shape=jax.ShapeDtypeStruct((2 * NPAD, HH), jnp.float32),
)


def _mid_body(a0_ref, a1_ref, b_ref, al_ref, w_ref, o_ref):
    z = jnp.concatenate([a0_ref[...], a1_ref[...]], axis=1) + b_ref[...]
    z = jnp.where(z >= 0, z, al_ref[...] * z)
    o_ref[...] = jnp.dot(z, w_ref[...], preferred_element_type=jnp.float32)


_mm_mid = pl.pallas_call(
    _mid_body,
    grid=(GRB, 2),
    in_specs=[
        pl.BlockSpec((RB, HH), lambda i, j: (i, 0)),
        pl.BlockSpec((RB, HH), lambda i, j: (i + GRB, 0)),
        pl.BlockSpec((1, H), lambda i, j: (0, 0)),
        pl.BlockSpec((1, H), lambda i, j: (0, 0)),
        pl.BlockSpec((H, HH), lambda i, j: (0, j)),
    ],
    out_specs=pl.BlockSpec((RB, HH), lambda i, j: (i + GRB * j, 0)),
    out_shape=jax.ShapeDtypeStruct((2 * NPAD, HH), jnp.float32),
)


def _fin_body(a0_ref, a1_ref, b_ref, al_ref, o_ref):
    z = jnp.concatenate([a0_ref[...], a1_ref[...]], axis=1) + b_ref[...]
    o_ref[...] = jnp.where(z >= 0, z, al_ref[...] * z)


_fin = pl.pallas_call(
    _fin_body,
    grid=(GRB,),
    in_specs=[
        pl.BlockSpec((RB, HH), lambda i: (i, 0)),
        pl.BlockSpec((RB, HH), lambda i: (i + GRB, 0)),
        pl.BlockSpec((1, H), lambda i: (0, 0)),
        pl.BlockSpec((1, H), lambda i: (0, 0)),
    ],
    out_specs=pl.BlockSpec((RB, H), lambda i: (i, 0)),
    out_shape=jax.ShapeDtypeStruct((NPAD, H), jnp.float32),
)


def _prep_edges(edge_index, edge_weight):
    loop = jnp.arange(N, dtype=edge_index.dtype)
    pad = EP - (E + N)
    rows = jnp.concatenate(
        [edge_index[0], loop, jnp.zeros((pad,), edge_index.dtype)])
    cols = jnp.concatenate(
        [edge_index[1], loop, jnp.zeros((pad,), edge_index.dtype)])
    ws = jnp.concatenate(
        [edge_weight, jnp.ones((N,), edge_weight.dtype),
         jnp.zeros((pad,), edge_weight.dtype)])
    wbits = jax.lax.bitcast_convert_type(ws, jnp.int32)
    packed = jnp.stack(
        [rows.reshape(NS * NCH, C), cols.reshape(NS * NCH, C),
         wbits.reshape(NS * NCH, C)], axis=1).reshape(NS * NCH * 3 * C)
    return packed


def _branch(x, edge_index, edge_weight, params):
    packed = _prep_edges(edge_index, edge_weight)
    dis = _dis_call(packed)
    zero = jnp.zeros((NPAD, HH), jnp.float32)
    xp = jnp.pad(x, ((0, NPAD - N), (0, 0)))

    (W0, b0, a0), (W1, b1, a1), (W2, b2, a2) = params
    h = _mm_first(xp, W0)
    acc = _edge_call(h, packed, dis, zero)
    h = _mm_mid(acc, acc, b0.reshape(1, H), a0.reshape(1, H), W1)
    acc = _edge_call(h, packed, dis, zero)
    h = _mm_mid(acc, acc, b1.reshape(1, H), a1.reshape(1, H), W2)
    acc = _edge_call(h, packed, dis, zero)
    return _fin(acc, acc, b2.reshape(1, H), a2.reshape(1, H))


def kernel(x1, edge_index1, edge_weight1, x2, edge_index2, edge_weight2,
           W1, b1, a1, W2, b2, a2, W1a, b1a, a1a, W2a, b2a, a2a,
           W1b, b1b, a1b, W2b, b2b, a2b):
    o1 = _branch(x1, edge_index1, edge_weight1,
                 ((W1, b1, a1), (W1a, b1a, a1a), (W1b, b1b, a1b)))
    o2 = _branch(x2, edge_index2, edge_weight2,
                 ((W2, b2, a2), (W2a, b2a, a2a), (W2b, b2b, a2b)))
    return jnp.concatenate([o1[:N], o2[:N]], axis=-1)


# speedup vs baseline: 1.0058x; 1.0058x over previous
import functools

import jax
import jax.numpy as jnp
from jax import lax
from jax.experimental import pallas as pl
from jax.experimental.pallas import tpu as pltpu
from jax.experimental.pallas import tpu_sc as plsc

N = 10000
E = 160000
H = 256
HH = 128
NS = 16
NPAD = 10240
RPT = NPAD // NS
C = 64
NBUF = 4
PD = 3
NCH = 168
ET = NCH * C
EP = NS * ET
RB = 640
GRB = NPAD // RB


def _rsqrt_v(x):
    i = plsc.bitcast(x, jnp.int32)
    y = plsc.bitcast(jnp.int32(0x5F3759DF) - (i >> 1), jnp.float32)
    for _ in range(3):
        y = y * (1.5 - 0.5 * x * y * y)
    return y


def _dis_body(pk_hbm, dis_hbm, ib, idxc, ewt, deg_t, deg_sh):
    c = lax.axis_index("c")
    s = lax.axis_index("s")

    @pl.when(c == 0)
    def _():
        def zb(i, _):
            deg_t[pl.ds(i * 16, 16)] = jnp.zeros((16,), jnp.float32)
            return 0
        lax.fori_loop(0, RPT // 16, zb, 0)
        pltpu.sync_copy(deg_t, deg_sh.at[pl.ds(s * RPT, RPT)])
        plsc.subcore_barrier()

        def chunk(k, _):
            sk = s * NCH + k
            pltpu.sync_copy(pk_hbm.at[pl.ds(sk * 3 * C, 3 * C)], ib)

            def grp(g, _):
                sl = pl.ds(g * 16, 16)
                idxc[sl] = ib[pl.ds(C + g * 16, 16)]
                ewt[sl] = plsc.bitcast(ib[pl.ds(2 * C + g * 16, 16)],
                                       jnp.float32)
                return 0
            lax.fori_loop(0, C // 16, grp, 0)
            pltpu.sync_copy(ewt, deg_sh.at[idxc], add=True)
            return 0
        lax.fori_loop(0, NCH, chunk, 0)
        plsc.subcore_barrier()

        pltpu.sync_copy(deg_sh.at[pl.ds(s * RPT, RPT)], deg_t)

        def rs(i, _):
            sl = pl.ds(i * 16, 16)
            deg_t[sl] = _rsqrt_v(deg_t[sl])
            return 0
        lax.fori_loop(0, RPT // 16, rs, 0)
        pltpu.sync_copy(deg_t, dis_hbm.at[pl.ds(s * RPT, RPT)])


_dis_call = pl.kernel(
    _dis_body,
    out_type=jax.ShapeDtypeStruct((NPAD,), jnp.float32),
    mesh=plsc.VectorSubcoreMesh(core_axis_name="c", subcore_axis_name="s"),
    scratch_types=[
        pltpu.VMEM((3 * C,), jnp.int32),
        pltpu.VMEM((C,), jnp.int32),
        pltpu.VMEM((C,), jnp.float32),
        pltpu.VMEM((RPT,), jnp.float32),
        pltpu.VMEM_SHARED((NPAD,), jnp.float32),
    ],
    compiler_params=pltpu.CompilerParams(needs_layout_passes=False),
)


def _edge_body(h_hbm, pk_hbm, dis_hbm, zero_hbm, acc_hbm, dis_t, *sc):
    c = lax.axis_index("c")
    s = lax.axis_index("s")
    ibs = sc[0:NBUF]
    gis = sc[NBUF:2 * NBUF]
    cis = sc[2 * NBUF:3 * NBUF]
    cfs = sc[3 * NBUF:4 * NBUF]
    gbs = sc[4 * NBUF:5 * NBUF]
    acc_sh = sc[5 * NBUF]
    isems = sc[5 * NBUF + 1:5 * NBUF + 1 + NBUF]
    gsems = sc[5 * NBUF + 1 + NBUF:5 * NBUF + 1 + 2 * NBUF]
    ssems = sc[5 * NBUF + 1 + 2 * NBUF:5 * NBUF + 1 + 3 * NBUF]

    pltpu.sync_copy(dis_hbm, dis_t)
    pltpu.sync_copy(zero_hbm.at[pl.ds(s * RPT, RPT)],
                    acc_sh.at[pl.ds(s * RPT, RPT)])
    plsc.subcore_barrier()

    def idx_copy(k, b):
        return pltpu.make_async_copy(
            pk_hbm.at[pl.ds((s * NCH + k) * 3 * C, 3 * C)], ibs[b], isems[b])

    def prep(b):
        ib, gi, ci, cf = ibs[b], gis[b], cis[b], cfs[b]

        def grp(g, _):
            sl = pl.ds(g * 16, 16)
            rv = ib[pl.ds(g * 16, 16)]
            cv = ib[pl.ds(C + g * 16, 16)]
            wv = plsc.bitcast(ib[pl.ds(2 * C + g * 16, 16)], jnp.float32)
            dr = plsc.load_gather(dis_t, [rv])
            dc = plsc.load_gather(dis_t, [cv])
            cf[sl] = wv * dr * dc
            gi[sl] = rv + c * NPAD
            ci[sl] = cv
            return 0
        lax.fori_loop(0, C // 16, grp, 0)

    def gather_copy(b):
        return pltpu.make_async_copy(h_hbm.at[gis[b]], gbs[b], gsems[b])

    def scale(b):
        gb, cf = gbs[b], cfs[b]

        def grp2(g, _):
            cbs = [plsc.load_gather(cf, [jnp.broadcast_to(g * 16 + j, (16,))])
                   for j in range(16)]
            for j in range(16):
                i = g * 16 + j
                for f in range(HH // 16):
                    fs = pl.ds(f * 16, 16)
                    gb[i, fs] = gb[i, fs] * cbs[j]
            return 0
        lax.fori_loop(0, C // 16, grp2, 0)

    def scatter_copy(b):
        return pltpu.async_copy(gbs[b], acc_sh.at[cis[b]], ssems[b], add=True)

    def scatter_wait(b):
        pltpu.make_async_copy(gbs[b], acc_sh.at[cis[b]], ssems[b]).wait()

    D = PD
    for j in range(D):
        idx_copy(j, j).start()
    for j in range(D):
        idx_copy(j, j).wait()
        prep(j)
        gather_copy(j).start()
    idx_copy(D, D).start()

    def outer(t, _):
        for b in range(NBUF):
            k = t * NBUF + b
            gather_copy(b).wait()
            scale(b)
            scatter_copy(b)
            nb = (b + D) % NBUF

            @pl.when(k + D < NCH)
            def _():
                @pl.when(k >= NBUF - D)
                def _():
                    scatter_wait(nb)
                idx_copy(k + D, nb).wait()
                prep(nb)
                gather_copy(nb).start()

                nb2 = (b + D + 1) % NBUF

                @pl.when(k + D + 1 < NCH)
                def _():
                    idx_copy(k + D + 1, nb2).start()
        return 0
    lax.fori_loop(0, NCH // NBUF, outer, 0)
    for j in range(NCH - D, NCH):
        scatter_wait(j % NBUF)
    plsc.subcore_barrier()

    pltpu.sync_copy(acc_sh.at[pl.ds(s * RPT, RPT)],
                    acc_hbm.at[pl.ds(c * NPAD + s * RPT, RPT)])


_edge_call = pl.kernel(
    _edge_body,
    out_type=jax.ShapeDtypeStruct((2 * NPAD, HH), jnp.float32),
    mesh=plsc.VectorSubcoreMesh(core_axis_name="c", subcore_axis_name="s"),
    scratch_types=(
        [pltpu.VMEM((NPAD,), jnp.float32)]
        + [pltpu.VMEM((3 * C,), jnp.int32) for _ in range(NBUF)]
        + [pltpu.VMEM((C,), jnp.int32) for _ in range(NBUF)]
        + [pltpu.VMEM((C,), jnp.int32) for _ in range(NBUF)]
        + [pltpu.VMEM((C,), jnp.float32) for _ in range(NBUF)]
        + [pltpu.VMEM((C, HH), jnp.float32) for _ in range(NBUF)]
        + [pltpu.VMEM_SHARED((NPAD, HH), jnp.float32)]
        + [pltpu.SemaphoreType.DMA for _ in range(3 * NBUF)]
    ),
    compiler_params=pltpu.CompilerParams(needs_layout_passes=False),
)


def _mm_first_body(x_ref, w_ref, o_ref):
    o_ref[...] = jnp.dot(x_ref[...], w_ref[...],
                         preferred_element_type=jnp.float32)


_mm_first = pl.pallas_call(
    _mm_first_body,
    grid=(GRB, 2),
    in_specs=[
        pl.BlockSpec((RB, H), lambda i, j: (i, 0)),
        pl.BlockSpec((H, HH), lambda i, j: (0, j)),
    ],
    out_specs=pl.BlockSpec((RB, HH), lambda i, j: (i + GRB * j, 0)),
    out_shape=jax.ShapeDtypeStruct((2 * NPAD, HH), jnp.float32),
)


def _mid_body(a0_ref, a1_ref, b_ref, al_ref, w_ref, o_ref):
    z = jnp.concatenate([a0_ref[...], a1_ref[...]], axis=1) + b_ref[...]
    z = jnp.where(z >= 0, z, al_ref[...] * z)
    o_ref[...] = jnp.dot(z, w_ref[...], preferred_element_type=jnp.float32)


_mm_mid = pl.pallas_call(
    _mid_body,
    grid=(GRB, 2),
    in_specs=[
        pl.BlockSpec((RB, HH), lambda i, j: (i, 0)),
        pl.BlockSpec((RB, HH), lambda i, j: (i + GRB, 0)),
        pl.BlockSpec((1, H), lambda i, j: (0, 0)),
        pl.BlockSpec((1, H), lambda i, j: (0, 0)),
        pl.BlockSpec((H, HH), lambda i, j: (0, j)),
    ],
    out_specs=pl.BlockSpec((RB, HH), lambda i, j: (i + GRB * j, 0)),
    out_shape=jax.ShapeDtypeStruct((2 * NPAD, HH), jnp.float32),
)


def _fin_body(a0_ref, a1_ref, b_ref, al_ref, o_ref):
    z = jnp.concatenate([a0_ref[...], a1_ref[...]], axis=1) + b_ref[...]
    o_ref[...] = jnp.where(z >= 0, z, al_ref[...] * z)


_fin = pl.pallas_call(
    _fin_body,
    grid=(GRB,),
    in_specs=[
        pl.BlockSpec((RB, HH), lambda i: (i, 0)),
        pl.BlockSpec((RB, HH), lambda i: (i + GRB, 0)),
        pl.BlockSpec((1, H), lambda i: (0, 0)),
        pl.BlockSpec((1, H), lambda i: (0, 0)),
    ],
    out_specs=pl.BlockSpec((RB, H), lambda i: (i, 0)),
    out_shape=jax.ShapeDtypeStruct((NPAD, H), jnp.float32),
)


def _prep_edges(edge_index, edge_weight):
    loop = jnp.arange(N, dtype=edge_index.dtype)
    pad = EP - (E + N)
    rows = jnp.concatenate(
        [edge_index[0], loop, jnp.zeros((pad,), edge_index.dtype)])
    cols = jnp.concatenate(
        [edge_index[1], loop, jnp.zeros((pad,), edge_index.dtype)])
    ws = jnp.concatenate(
        [edge_weight, jnp.ones((N,), edge_weight.dtype),
         jnp.zeros((pad,), edge_weight.dtype)])
    wbits = jax.lax.bitcast_convert_type(ws, jnp.int32)
    packed = jnp.stack(
        [rows.reshape(NS * NCH, C), cols.reshape(NS * NCH, C),
         wbits.reshape(NS * NCH, C)], axis=1).reshape(NS * NCH * 3 * C)
    return packed


def _branch(x, edge_index, edge_weight, params):
    packed = _prep_edges(edge_index, edge_weight)
    dis = _dis_call(packed)
    zero = jnp.zeros((NPAD, HH), jnp.float32)
    xp = jnp.pad(x, ((0, NPAD - N), (0, 0)))

    (W0, b0, a0), (W1, b1, a1), (W2, b2, a2) = params
    h = _mm_first(xp, W0)
    acc = _edge_call(h, packed, dis, zero)
    h = _mm_mid(acc, acc, b0.reshape(1, H), a0.reshape(1, H), W1)
    acc = _edge_call(h, packed, dis, zero)
    h = _mm_mid(acc, acc, b1.reshape(1, H), a1.reshape(1, H), W2)
    acc = _edge_call(h, packed, dis, zero)
    return _fin(acc, acc, b2.reshape(1, H), a2.reshape(1, H))


def kernel(x1, edge_index1, edge_weight1, x2, edge_index2, edge_weight2,
           W1, b1, a1, W2, b2, a2, W1a, b1a, a1a, W2a, b2a, a2a,
           W1b, b1b, a1b, W2b, b2b, a2b):
    o1 = _branch(x1, edge_index1, edge_weight1,
                 ((W1, b1, a1), (W1a, b1a, a1a), (W1b, b1b, a1b)))
    o2 = _branch(x2, edge_index2, edge_weight2,
                 ((W2, b2, a2), (W2a, b2a, a2a), (W2b, b2b, a2b)))
    return jnp.concatenate([o1[:N], o2[:N]], axis=-1)
